# Initial kernel scaffold; baseline (speedup 1.0000x reference)
import functools

import jax
import jax.numpy as jnp
from jax import lax
from jax.experimental import pallas as pl
from jax.experimental.pallas import tpu as pltpu
from jax.experimental.pallas import tpu_sc as plsc

B = 16
S = 2048
H = 32
KVH = 8
DH = 128
SLOTS = 32768
SCALE = 0.08838834764831845
GROUP = H // KVH

C_SC = 32
ITEMS_PER_B = S // C_SC
NW = 32
ITEMS = B * ITEMS_PER_B
C_TC = 256
NCHUNK = S // C_TC

NEG = jnp.float32(-1e30)


def _sc_gather(k_cache, v_cache, active_slots, lens):
    mesh = plsc.VectorSubcoreMesh(
        core_axis_name="c", subcore_axis_name="s", num_cores=2, num_subcores=16
    )
    out_sd = jax.ShapeDtypeStruct((B, KVH, S, DH), jnp.float32)

    @functools.partial(
        pl.kernel,
        out_type=[out_sd, out_sd],
        mesh=mesh,
        scratch_types=[
            pltpu.VMEM((B,), jnp.int32),
            pltpu.VMEM((C_SC,), jnp.int32),
            pltpu.VMEM((C_SC, KVH, DH), jnp.float32),
            pltpu.VMEM((C_SC, KVH, DH), jnp.float32),
            pltpu.SemaphoreType.DMA,
            pltpu.SemaphoreType.DMA,
        ],
    )
    def sc_kernel(kc, vc, slots, lens_h, gk, gv,
                  lens_v, idx_v, krows, vrows, sem_k, sem_v):
        wid = lax.axis_index("s") * 2 + lax.axis_index("c")
        pltpu.sync_copy(lens_h, lens_v)
        lane = lax.broadcasted_iota(jnp.int32, (16,), 0)

        def body(it, carry):
            item = it * NW + wid
            b = item // ITEMS_PER_B
            j = item % ITEMS_PER_B
            lb = jnp.max(jnp.where(lane == b, lens_v[...], 0), axis=0)

            @pl.when(j * C_SC < lb)
            def _():
                pltpu.sync_copy(slots.at[b, pl.ds(j * C_SC, C_SC)], idx_v)
                ck = pltpu.async_copy(kc.at[idx_v], krows, sem_k)
                cv = pltpu.async_copy(vc.at[idx_v], vrows, sem_v)
                ck.wait()
                cv.wait()
                for g in range(KVH):
                    pltpu.sync_copy(krows.at[:, g, :],
                                    gk.at[b, g, pl.ds(j * C_SC, C_SC)])
                    pltpu.sync_copy(vrows.at[:, g, :],
                                    gv.at[b, g, pl.ds(j * C_SC, C_SC)])

            return carry

        lax.fori_loop(0, ITEMS // NW, body, 0)

    return sc_kernel(k_cache, v_cache, active_slots, lens)


def _tc_body(lens_ref, nactm1_ref, q_ref, k_ref, v_ref, kn_ref, vn_ref,
             slots_ref, sm_ref, o_ref, m_scr, l_scr, acc_scr):
    b = pl.program_id(0)
    c = pl.program_id(2)

    @pl.when(c == 0)
    def _():
        m_scr[...] = jnp.full((GROUP, DH), NEG, jnp.float32)
        l_scr[...] = jnp.zeros((GROUP, DH), jnp.float32)
        acc_scr[...] = jnp.zeros((GROUP, DH), jnp.float32)

    @pl.when(c <= nactm1_ref[b])
    def _():
        lb = lens_ref[b]
        qg = q_ref[0, 0]
        kk = k_ref[0, 0]
        vv = v_ref[0, 0]
        kn_g = kn_ref[0]
        vn_g = vn_ref[0]
        sm_col = sm_ref[...]
        slots_row = slots_ref[0, 0]

        s = lax.dot_general(qg, kk, (((1,), (1,)), ((), ())),
                            preferred_element_type=jnp.float32)
        s = s * SCALE

        oh = (sm_col == slots_row).astype(jnp.float32)
        newmask = jnp.max(oh, axis=0, keepdims=True)
        cand = lax.dot_general(qg, kn_g, (((1,), (1,)), ((), ())),
                               preferred_element_type=jnp.float32) * SCALE
        s_patch = lax.dot_general(cand, oh, (((1,), (0,)), ((), ())),
                                  preferred_element_type=jnp.float32)
        s = jnp.where(newmask > 0, s_patch, s)

        pos = c * C_TC + lax.broadcasted_iota(jnp.int32, (1, C_TC), 1)
        s = jnp.where(pos < lb, s, NEG)

        m_old = m_scr[:, 0:1]
        m_new = jnp.maximum(m_old, jnp.max(s, axis=1, keepdims=True))
        alpha = jnp.exp(m_old - m_new)
        p = jnp.exp(s - m_new)

        p_old = p * (1.0 - newmask)
        pn = lax.dot_general(p * newmask, oh, (((1,), (1,)), ((), ())),
                             preferred_element_type=jnp.float32)

        pos_col = c * C_TC + lax.broadcasted_iota(jnp.int32, (C_TC, 1), 0)
        v_use = jnp.where(pos_col < lb, vv, 0.0)

        acc = acc_scr[...] * alpha
        acc = acc + lax.dot_general(p_old, v_use, (((1,), (0,)), ((), ())),
                                    preferred_element_type=jnp.float32)
        acc = acc + lax.dot_general(pn, vn_g, (((1,), (0,)), ((), ())),
                                    preferred_element_type=jnp.float32)
        l_new = l_scr[:, 0:1] * alpha + jnp.sum(p, axis=1, keepdims=True)

        m_scr[...] = jnp.broadcast_to(m_new, (GROUP, DH))
        l_scr[...] = jnp.broadcast_to(l_new, (GROUP, DH))
        acc_scr[...] = acc
        o_ref[0, 0] = acc / l_new


def _tc_attend(q4, gk, gv, kn_t, vn_t, slots4, sm2, lens, nactm1):
    def q_map(b, g, c, lens_ref, nactm1_ref):
        return (b, g, 0, 0)

    def kv_map(b, g, c, lens_ref, nactm1_ref):
        return (b, g, jnp.minimum(c, nactm1_ref[b]), 0)

    def kn_map(b, g, c, lens_ref, nactm1_ref):
        return (g, 0, 0)

    def slots_map(b, g, c, lens_ref, nactm1_ref):
        return (b, jnp.minimum(c, nactm1_ref[b]), 0, 0)

    def sm_map(b, g, c, lens_ref, nactm1_ref):
        return (0, 0)

    grid_spec = pltpu.PrefetchScalarGridSpec(
        num_scalar_prefetch=2,
        grid=(B, KVH, NCHUNK),
        in_specs=[
            pl.BlockSpec((1, 1, GROUP, DH), q_map),
            pl.BlockSpec((1, 1, C_TC, DH), kv_map),
            pl.BlockSpec((1, 1, C_TC, DH), kv_map),
            pl.BlockSpec((1, B, DH), kn_map),
            pl.BlockSpec((1, B, DH), kn_map),
            pl.BlockSpec((1, 1, 1, C_TC), slots_map),
            pl.BlockSpec((B, 1), sm_map),
        ],
        out_specs=pl.BlockSpec((1, 1, GROUP, DH), q_map),
        scratch_shapes=[
            pltpu.VMEM((GROUP, DH), jnp.float32),
            pltpu.VMEM((GROUP, DH), jnp.float32),
            pltpu.VMEM((GROUP, DH), jnp.float32),
        ],
    )
    return pl.pallas_call(
        _tc_body,
        grid_spec=grid_spec,
        out_shape=jax.ShapeDtypeStruct((B, KVH, GROUP, DH), jnp.float32),
    )(lens, nactm1, q4, gk, gv, kn_t, vn_t, slots4, sm2)


def kernel(q, k, v, k_cache, v_cache, slot_mapping, active_slots, context_lens):
    lens = jnp.maximum(context_lens, 1).astype(jnp.int32)
    nactm1 = (lens - 1) // C_TC

    gk, gv = _sc_gather(k_cache, v_cache, active_slots, lens)

    q4 = q.reshape(B, KVH, GROUP, DH)
    kn_t = jnp.transpose(k, (1, 0, 2))
    vn_t = jnp.transpose(v, (1, 0, 2))
    slots4 = active_slots.reshape(B, NCHUNK, 1, C_TC)
    sm2 = slot_mapping.astype(jnp.int32).reshape(B, 1)

    o4 = _tc_attend(q4, gk, gv, kn_t, vn_t, slots4, sm2, lens, nactm1)
    return o4.reshape(B, H, DH)


# speedup vs baseline: 2.4568x; 2.4568x over previous
import functools

import jax
import jax.numpy as jnp
from jax import lax
from jax.experimental import pallas as pl
from jax.experimental.pallas import tpu as pltpu
from jax.experimental.pallas import tpu_sc as plsc

B = 16
S = 2048
H = 32
KVH = 8
DH = 128
SLOTS = 32768
SCALE = 0.08838834764831845
GROUP = H // KVH

C_SC = 32
ITEMS_PER_B = S // C_SC
NW = 32
ITEMS = B * ITEMS_PER_B
C_TC = 256
NCHUNK = S // C_TC

NEG = -1e30


def _sc_gather(k_cache, v_cache, active_slots, lens):
    mesh = plsc.VectorSubcoreMesh(
        core_axis_name="c", subcore_axis_name="s", num_cores=2, num_subcores=16
    )
    out_sd = jax.ShapeDtypeStruct((B, KVH, S, DH), jnp.float32)

    @functools.partial(
        pl.kernel,
        out_type=[out_sd, out_sd],
        mesh=mesh,
        scratch_types=[
            pltpu.VMEM((B,), jnp.int32),
            pltpu.VMEM((C_SC,), jnp.int32),
            pltpu.VMEM((C_SC, KVH, DH), jnp.float32),
            pltpu.VMEM((C_SC, KVH, DH), jnp.float32),
            pltpu.SemaphoreType.DMA,
            pltpu.SemaphoreType.DMA,
        ],
    )
    def sc_kernel(kc, vc, slots, lens_h, gk, gv,
                  lens_v, idx_v, krows, vrows, sem_k, sem_v):
        wid = lax.axis_index("s") * 2 + lax.axis_index("c")
        pltpu.sync_copy(lens_h, lens_v)
        lens_vec = lens_v[...]
        for b in range(B):
            lb = lens_vec[b]
            for r in range(ITEMS_PER_B // NW):
                j = r * NW + wid

                @pl.when(j * C_SC < lb)
                def _(b=b, j=j):
                    pltpu.sync_copy(slots.at[b, pl.ds(j * C_SC, C_SC)], idx_v)
                    ck = pltpu.async_copy(kc.at[idx_v], krows, sem_k)
                    cv = pltpu.async_copy(vc.at[idx_v], vrows, sem_v)
                    ck.wait()
                    cv.wait()
                    for g in range(KVH):
                        pltpu.sync_copy(krows.at[:, g, :],
                                        gk.at[b, g, pl.ds(j * C_SC, C_SC)])
                        pltpu.sync_copy(vrows.at[:, g, :],
                                        gv.at[b, g, pl.ds(j * C_SC, C_SC)])

    return sc_kernel(k_cache, v_cache, active_slots, lens)


def _tc_body(lens_ref, nactm1_ref, q_ref, k_ref, v_ref, kn_ref, vn_ref,
             slots_ref, sm_ref, o_ref, m_scr, l_scr, acc_scr):
    b = pl.program_id(0)
    c = pl.program_id(2)

    @pl.when(c == 0)
    def _():
        m_scr[...] = jnp.full((GROUP, DH), NEG, jnp.float32)
        l_scr[...] = jnp.zeros((GROUP, DH), jnp.float32)
        acc_scr[...] = jnp.zeros((GROUP, DH), jnp.float32)

    @pl.when(c <= nactm1_ref[b])
    def _():
        lb = lens_ref[b]
        qg = q_ref[0, 0]
        kk = k_ref[0, 0]
        vv = v_ref[0, 0]
        kn_g = kn_ref[0]
        vn_g = vn_ref[0]
        sm_col = sm_ref[...]
        slots_row = slots_ref[0, 0]

        s = lax.dot_general(qg, kk, (((1,), (1,)), ((), ())),
                            preferred_element_type=jnp.float32)
        s = s * SCALE

        oh = (sm_col == slots_row).astype(jnp.float32)
        newmask = jnp.max(oh, axis=0, keepdims=True)
        cand = lax.dot_general(qg, kn_g, (((1,), (1,)), ((), ())),
                               preferred_element_type=jnp.float32) * SCALE
        s_patch = lax.dot_general(cand, oh, (((1,), (0,)), ((), ())),
                                  preferred_element_type=jnp.float32)
        s = jnp.where(newmask > 0, s_patch, s)

        pos = c * C_TC + lax.broadcasted_iota(jnp.int32, (1, C_TC), 1)
        s = jnp.where(pos < lb, s, NEG)

        m_old = m_scr[:, 0:1]
        m_new = jnp.maximum(m_old, jnp.max(s, axis=1, keepdims=True))
        alpha = jnp.exp(m_old - m_new)
        p = jnp.exp(s - m_new)

        p_old = p * (1.0 - newmask)
        pn = lax.dot_general(p * newmask, oh, (((1,), (1,)), ((), ())),
                             preferred_element_type=jnp.float32)

        pos_col = c * C_TC + lax.broadcasted_iota(jnp.int32, (C_TC, 1), 0)
        v_use = jnp.where(pos_col < lb, vv, 0.0)

        acc = acc_scr[...] * alpha
        acc = acc + lax.dot_general(p_old, v_use, (((1,), (0,)), ((), ())),
                                    preferred_element_type=jnp.float32)
        acc = acc + lax.dot_general(pn, vn_g, (((1,), (0,)), ((), ())),
                                    preferred_element_type=jnp.float32)
        l_new = l_scr[:, 0:1] * alpha + jnp.sum(p, axis=1, keepdims=True)

        m_scr[...] = jnp.broadcast_to(m_new, (GROUP, DH))
        l_scr[...] = jnp.broadcast_to(l_new, (GROUP, DH))
        acc_scr[...] = acc
        o_ref[0, 0] = acc / l_new


def _tc_attend(q4, gk, gv, kn_t, vn_t, slots4, sm2, lens, nactm1):
    def q_map(b, g, c, lens_ref, nactm1_ref):
        return (b, g, 0, 0)

    def kv_map(b, g, c, lens_ref, nactm1_ref):
        return (b, g, jnp.minimum(c, nactm1_ref[b]), 0)

    def kn_map(b, g, c, lens_ref, nactm1_ref):
        return (g, 0, 0)

    def slots_map(b, g, c, lens_ref, nactm1_ref):
        return (b, jnp.minimum(c, nactm1_ref[b]), 0, 0)

    def sm_map(b, g, c, lens_ref, nactm1_ref):
        return (0, 0)

    grid_spec = pltpu.PrefetchScalarGridSpec(
        num_scalar_prefetch=2,
        grid=(B, KVH, NCHUNK),
        in_specs=[
            pl.BlockSpec((1, 1, GROUP, DH), q_map),
            pl.BlockSpec((1, 1, C_TC, DH), kv_map),
            pl.BlockSpec((1, 1, C_TC, DH), kv_map),
            pl.BlockSpec((1, B, DH), kn_map),
            pl.BlockSpec((1, B, DH), kn_map),
            pl.BlockSpec((1, 1, 1, C_TC), slots_map),
            pl.BlockSpec((B, 1), sm_map),
        ],
        out_specs=pl.BlockSpec((1, 1, GROUP, DH), q_map),
        scratch_shapes=[
            pltpu.VMEM((GROUP, DH), jnp.float32),
            pltpu.VMEM((GROUP, DH), jnp.float32),
            pltpu.VMEM((GROUP, DH), jnp.float32),
        ],
    )
    return pl.pallas_call(
        _tc_body,
        grid_spec=grid_spec,
        out_shape=jax.ShapeDtypeStruct((B, KVH, GROUP, DH), jnp.float32),
    )(lens, nactm1, q4, gk, gv, kn_t, vn_t, slots4, sm2)


def kernel(q, k, v, k_cache, v_cache, slot_mapping, active_slots, context_lens):
    lens = jnp.maximum(context_lens, 1).astype(jnp.int32)
    nactm1 = (lens - 1) // C_TC

    gk, gv = _sc_gather(k_cache, v_cache, active_slots, lens)

    q4 = q.reshape(B, KVH, GROUP, DH)
    kn_t = jnp.transpose(k, (1, 0, 2))
    vn_t = jnp.transpose(v, (1, 0, 2))
    slots4 = active_slots.reshape(B, NCHUNK, 1, C_TC)
    sm2 = slot_mapping.astype(jnp.int32).reshape(B, 1)

    o4 = _tc_attend(q4, gk, gv, kn_t, vn_t, slots4, sm2, lens, nactm1)
    return o4.reshape(B, H, DH)


# speedup vs baseline: 5.9293x; 2.4135x over previous
import functools

import jax
import jax.numpy as jnp
from jax import lax
from jax.experimental import pallas as pl
from jax.experimental.pallas import tpu as pltpu
from jax.experimental.pallas import tpu_sc as plsc

B = 16
S = 2048
H = 32
KVH = 8
DH = 128
SLOTS = 32768
SCALE = 0.08838834764831845
GROUP = H // KVH

C_SC = 32
ITEMS_PER_B = S // C_SC
NW = 32
ITEMS = B * ITEMS_PER_B
C_TC = 256
NCHUNK = S // C_TC

NEG = -1e30


def _sc_gather(k_cache, v_cache, active_slots, lens):
    mesh = plsc.VectorSubcoreMesh(
        core_axis_name="c", subcore_axis_name="s", num_cores=2, num_subcores=16
    )
    out_sd = jax.ShapeDtypeStruct((B, KVH, S, DH), jnp.float32)

    @functools.partial(
        pl.kernel,
        out_type=[out_sd, out_sd],
        mesh=mesh,
        scratch_types=[
            pltpu.VMEM((B,), jnp.int32),
            pltpu.VMEM((C_SC,), jnp.int32),
            pltpu.VMEM((C_SC, KVH, DH), jnp.float32),
            pltpu.VMEM((C_SC, KVH, DH), jnp.float32),
            pltpu.SemaphoreType.DMA,
            pltpu.SemaphoreType.DMA,
        ],
    )
    def sc_kernel(kc, vc, slots, lens_h, gk, gv,
                  lens_v, idx_v, krows, vrows, sem_k, sem_v):
        wid = lax.axis_index("s") * 2 + lax.axis_index("c")
        pltpu.sync_copy(lens_h, lens_v)
        lens_vec = lens_v[...]
        for b in range(B):
            lb = lens_vec[b]
            for r in range(ITEMS_PER_B // NW):
                j = r * NW + wid

                @pl.when(j * C_SC < lb)
                def _(b=b, j=j):
                    pltpu.sync_copy(slots.at[b, pl.ds(j * C_SC, C_SC)], idx_v)
                    ck = pltpu.async_copy(kc.at[idx_v], krows, sem_k)
                    cv = pltpu.async_copy(vc.at[idx_v], vrows, sem_v)
                    ck.wait()
                    cv.wait()
                    for g in range(KVH):
                        pltpu.sync_copy(krows.at[:, g, :],
                                        gk.at[b, g, pl.ds(j * C_SC, C_SC)])
                        pltpu.sync_copy(vrows.at[:, g, :],
                                        gv.at[b, g, pl.ds(j * C_SC, C_SC)])

    return sc_kernel(k_cache, v_cache, active_slots, lens)


W = KVH * C_TC


def _tc_body(lens_ref, nactm1_ref, q_ref, k_ref, v_ref, kn_ref, vn_ref,
             slotsr_ref, slotsc_ref, smc_ref, smr_ref, o_ref,
             m_scr, l_scr, acc_scr, cnt_scr):
    b = pl.program_id(0)
    c = pl.program_id(1)

    @pl.when(c == 0)
    def _():
        m_scr[...] = jnp.full((H, DH), NEG, jnp.float32)
        l_scr[...] = jnp.zeros((H, DH), jnp.float32)
        acc_scr[...] = jnp.zeros((H, DH), jnp.float32)
        cnt_scr[...] = jnp.zeros((B, DH), jnp.float32)

    @pl.when(c <= nactm1_ref[b])
    def _():
        lb = lens_ref[b]
        q_all = q_ref[0]
        kflat = k_ref[0].reshape(W, DH)
        vflat = v_ref[0].reshape(W, DH)
        sm_c = smc_ref[...]
        sm_r = smr_ref[...]
        slots_row = slotsr_ref[0, 0]
        slots_col = slotsc_ref[0, 0]

        match16 = sm_c == slots_row
        validr = (c * C_TC + lax.broadcasted_iota(jnp.int32, (1, C_TC), 1)) < lb
        cnt_add = jnp.sum(jnp.where(match16 & validr, 1.0, 0.0),
                          axis=1, keepdims=True)
        cnt_scr[...] = cnt_scr[...] + jnp.broadcast_to(cnt_add, (B, DH))

        newr = jnp.max(match16.astype(jnp.float32), axis=0, keepdims=True)
        keep_r = jnp.logical_and(validr, newr < 0.5)
        keep_row = jnp.concatenate([keep_r] * KVH, axis=1)
        hg = lax.broadcasted_iota(jnp.int32, (H, 1), 0) // GROUP
        jg = lax.broadcasted_iota(jnp.int32, (1, W), 1) // C_TC
        smask = jnp.logical_and(hg == jg, keep_row)

        matchc = jnp.max((slots_col == sm_r).astype(jnp.float32),
                         axis=1, keepdims=True)
        validc = (c * C_TC + lax.broadcasted_iota(jnp.int32, (C_TC, 1), 0)) < lb
        keep_c = jnp.logical_and(validc, matchc < 0.5)
        keep_col = jnp.broadcast_to(keep_c[None], (KVH, C_TC, 1)).reshape(W, 1)
        v_use = jnp.where(keep_col, vflat, 0.0)

        s = lax.dot_general(q_all, kflat, (((1,), (1,)), ((), ())),
                            preferred_element_type=jnp.float32) * SCALE
        s = jnp.where(smask, s, NEG)

        m_old = m_scr[:, 0:1]
        m_new = jnp.maximum(m_old, jnp.max(s, axis=1, keepdims=True))
        alpha = jnp.exp(m_old - m_new)
        p = jnp.where(smask, jnp.exp(s - m_new), 0.0)

        l_new = l_scr[:, 0:1] * alpha + jnp.sum(p, axis=1, keepdims=True)
        acc = acc_scr[...] * alpha + lax.dot_general(
            p, v_use, (((1,), (0,)), ((), ())),
            preferred_element_type=jnp.float32)

        m_scr[...] = jnp.broadcast_to(m_new, (H, DH))
        l_scr[...] = jnp.broadcast_to(l_new, (H, DH))
        acc_scr[...] = acc

    @pl.when(c == nactm1_ref[b])
    def _():
        q_all = q_ref[0]
        knf = kn_ref[...].reshape(KVH * B, DH)
        vnf = vn_ref[...].reshape(KVH * B, DH)
        cand = lax.dot_general(q_all, knf, (((1,), (1,)), ((), ())),
                               preferred_element_type=jnp.float32) * SCALE
        hg = lax.broadcasted_iota(jnp.int32, (H, 1), 0) // GROUP
        rg = lax.broadcasted_iota(jnp.int32, (1, KVH * B), 1) // B
        cand = jnp.where(hg == rg, cand, NEG)

        m_a = m_scr[:, 0:1]
        m_fin = jnp.maximum(m_a, jnp.max(cand, axis=1, keepdims=True))
        e_b = jnp.exp(cand - m_fin)

        cnt = cnt_scr[:, 0:1]
        cnt_w = jnp.broadcast_to(cnt[None], (KVH, B, 1)).reshape(KVH * B, 1)
        l_b = lax.dot_general(e_b, cnt_w, (((1,), (0,)), ((), ())),
                              preferred_element_type=jnp.float32)
        acc_b = lax.dot_general(e_b, vnf * cnt_w, (((1,), (0,)), ((), ())),
                                preferred_element_type=jnp.float32)

        alpha_a = jnp.exp(m_a - m_fin)
        l_fin = l_scr[:, 0:1] * alpha_a + l_b
        acc_fin = acc_scr[...] * alpha_a + acc_b
        o_ref[0] = acc_fin / l_fin


def _tc_attend(q, gk, gv, kn_t, vn_t, slots4, slots4c, sm_c, sm_r, lens, nactm1):
    def q_map(b, c, lens_ref, nactm1_ref):
        return (b, 0, 0)

    def kv_map(b, c, lens_ref, nactm1_ref):
        return (b, 0, jnp.minimum(c, nactm1_ref[b]), 0)

    def kn_map(b, c, lens_ref, nactm1_ref):
        return (0, 0, 0)

    def slots_map(b, c, lens_ref, nactm1_ref):
        return (b, jnp.minimum(c, nactm1_ref[b]), 0, 0)

    def sm_map(b, c, lens_ref, nactm1_ref):
        return (0, 0)

    grid_spec = pltpu.PrefetchScalarGridSpec(
        num_scalar_prefetch=2,
        grid=(B, NCHUNK),
        in_specs=[
            pl.BlockSpec((1, H, DH), q_map),
            pl.BlockSpec((1, KVH, C_TC, DH), kv_map),
            pl.BlockSpec((1, KVH, C_TC, DH), kv_map),
            pl.BlockSpec((KVH, B, DH), kn_map),
            pl.BlockSpec((KVH, B, DH), kn_map),
            pl.BlockSpec((1, 1, 1, C_TC), slots_map),
            pl.BlockSpec((1, 1, C_TC, 1), slots_map),
            pl.BlockSpec((B, 1), sm_map),
            pl.BlockSpec((1, B), sm_map),
        ],
        out_specs=pl.BlockSpec((1, H, DH), q_map),
        scratch_shapes=[
            pltpu.VMEM((H, DH), jnp.float32),
            pltpu.VMEM((H, DH), jnp.float32),
            pltpu.VMEM((H, DH), jnp.float32),
            pltpu.VMEM((B, DH), jnp.float32),
        ],
    )
    return pl.pallas_call(
        _tc_body,
        grid_spec=grid_spec,
        out_shape=jax.ShapeDtypeStruct((B, H, DH), jnp.float32),
    )(lens, nactm1, q, gk, gv, kn_t, vn_t, slots4, slots4c, sm_c, sm_r)


def kernel(q, k, v, k_cache, v_cache, slot_mapping, active_slots, context_lens):
    lens = jnp.maximum(context_lens, 1).astype(jnp.int32)
    nactm1 = (lens - 1) // C_TC

    gk, gv = _sc_gather(k_cache, v_cache, active_slots, lens)

    kn_t = jnp.transpose(k, (1, 0, 2))
    vn_t = jnp.transpose(v, (1, 0, 2))
    slots4 = active_slots.reshape(B, NCHUNK, 1, C_TC)
    slots4c = active_slots.reshape(B, NCHUNK, C_TC, 1)
    sm_i = slot_mapping.astype(jnp.int32)
    sm_c = sm_i.reshape(B, 1)
    sm_r = sm_i.reshape(1, B)

    return _tc_attend(q, gk, gv, kn_t, vn_t, slots4, slots4c, sm_c, sm_r,
                      lens, nactm1)
